# baseline (speedup 1.0000x reference)
import functools

import jax
import jax.numpy as jnp
from jax import lax
from jax.experimental import pallas as pl
from jax.experimental.pallas import tpu as pltpu
from jax.experimental.pallas import tpu_sc as plsc

D = 1024
B = 4 * 8192
NC, NS = 2, 16
NW = NC * NS
BPW = B // NW
C = 40
NBUF = 3
NFCH = BPW // C
RING = NFCH // NBUF
TAIL = NFCH - RING * NBUF
REM = BPW - NFCH * C


def _gather(table, idx):
  mesh = plsc.VectorSubcoreMesh(core_axis_name="c", subcore_axis_name="s")

  @functools.partial(
      pl.kernel,
      out_type=jax.ShapeDtypeStruct((B, D), jnp.float32),
      mesh=mesh,
      scratch_types=[
          pltpu.VMEM((BPW,), jnp.int32),
          [pltpu.VMEM((C, D), jnp.float32) for _ in range(NBUF)],
          [pltpu.SemaphoreType.DMA for _ in range(NBUF)],
          [pltpu.SemaphoreType.DMA for _ in range(NBUF)],
      ],
  )
  def k(table_hbm, idx_hbm, out_hbm, idx_v, rows, si, so):
    wid = lax.axis_index("s") * NC + lax.axis_index("c")
    base = wid * BPW
    pltpu.sync_copy(idx_hbm.at[pl.ds(base, BPW)], idx_v)

    def gather(j, b, n=C):
      pltpu.async_copy(table_hbm.at[idx_v.at[pl.ds(j * C, n)]],
                       rows[b].at[pl.ds(0, n)], si[b])

    def put(j, b, n=C):
      pltpu.async_copy(rows[b].at[pl.ds(0, n)],
                       out_hbm.at[pl.ds(base + j * C, n)], so[b])

    def wait(b, sem, n=C):
      pltpu.make_async_copy(table_hbm.at[pl.ds(0, n)],
                            rows[b].at[pl.ds(0, n)], sem[b]).wait()

    for b in range(NBUF):
      gather(b, b)

    @pl.loop(0, RING - 1)
    def _(i):
      j = i * NBUF
      for b in range(NBUF):
        wait(b, si)
        put(j + b, b)
      for b in range(NBUF):
        wait(b, so)
        gather(j + NBUF + b, b)

    jlast = (RING - 1) * NBUF
    for b in range(NBUF):
      wait(b, si)
      put(jlast + b, b)

    nxt = 0
    for t in range(TAIL):
      b = nxt % NBUF
      wait(b, so)
      gather(RING * NBUF + t, b)
      wait(b, si)
      put(RING * NBUF + t, b)
      nxt += 1
    if REM:
      b = nxt % NBUF
      wait(b, so)
      gather(NFCH, b, REM)
      wait(b, si, REM)
      put(NFCH, b, REM)

    for b in range(NBUF):
      if REM and b == nxt % NBUF:
        wait(b, so, REM)
      else:
        wait(b, so)

  return k(table, idx)


@jax.jit
def kernel(x, embedding_weight):
  idx = x.reshape(-1).astype(jnp.int32)
  out = _gather(embedding_weight, idx)
  return out.reshape(x.shape + (D,))
